# baseline (speedup 1.0000x reference)
import functools

import jax
import jax.numpy as jnp
from jax.experimental import pallas as pl
from jax.experimental.pallas import tpu as pltpu

_NEG = -1e30


def _acgnn_body(x_ref, adj_ref, win_ref, bin_ref,
                wlr1_ref, blr1t_ref, asc1_ref, lo1_ref, hi1_ref, kap1_ref,
                cb1_ref, g1_ref, be1_ref,
                wlr2_ref, blr2t_ref, asc2_ref, lo2_ref, hi2_ref, kap2_ref,
                cb2_ref, g2_ref, be2_ref,
                wac_ref, ba_ref, bc_ref, out_ref, *, heads, hd, n, gpb):
    hc = heads * hd

    adjbs = [jnp.where(adj_ref[g] > 0.0, 0.0, _NEG) for g in range(gpb)]

    def ln_relu(v, gamma, beta, eps=1e-5):
        mu = jnp.mean(v, axis=-1, keepdims=True)
        d = v - mu
        var = jnp.mean(d * d, axis=-1, keepdims=True)
        return jnp.maximum(d * jax.lax.rsqrt(var + eps) * gamma + beta, 0.0)

    def gat(hstk, wlr_ref, blrt_ref, asc_ref, lo_ref, hi_ref, kap_ref,
            cb_ref):
        gT = jax.lax.dot_general(wlr_ref[...], hstk, (((0,), (1,)), ((), ())),
                                 preferred_element_type=jnp.float32)
        gT = gT + blrt_ref[...]
        glT = gT[0:hc, :]
        grT = gT[hc:2 * hc, :]
        asc = asc_ref[...]
        glTs = (glT * asc).astype(jnp.bfloat16)
        grTs = (grT * asc).astype(jnp.bfloat16)
        clo = lo_ref[...].astype(jnp.bfloat16)
        chi = hi_ref[...].astype(jnp.bfloat16)
        gr3 = grTs[:, :, None]

        es = [None] * (gpb * heads)
        hgrp = 4
        for hp in range(heads // hgrp):
            accs = [None] * (hgrp * gpb)
            for c in range(hd):
                for hh in range(hgrp):
                    head = hgrp * hp + hh
                    sl = head * hd + c
                    l3 = clo[sl:sl + 1, :][:, :, None]
                    h3 = chi[sl:sl + 1, :][:, :, None]
                    for g in range(gpb):
                        j0 = g * n
                        u3 = gr3[sl:sl + 1, j0:j0 + n] \
                            + glTs[sl:sl + 1, None, j0:j0 + n]
                        r3 = jnp.clip(u3, l3, h3)
                        k = hh * gpb + g
                        accs[k] = r3 if accs[k] is None else accs[k] + r3
            for hh in range(hgrp):
                head = hgrp * hp + hh
                lo = head * hd
                for g in range(gpb):
                    j0 = g * n
                    lk = jax.lax.dot_general(
                        kap_ref[lo:lo + hd, :], glT[lo:lo + hd, j0:j0 + n],
                        (((0,), (0,)), ((), ())),
                        preferred_element_type=jnp.float32)
                    es[g * heads + head] = (
                        accs[hh * gpb + g][0].astype(jnp.float32)
                        + lk + adjbs[g])
        ee = jnp.concatenate(es, axis=0)
        ee = ee - jnp.max(ee, axis=-1, keepdims=True)
        p = jnp.exp(ee)
        p = p * pl.reciprocal(jnp.sum(p, axis=-1, keepdims=True),
                              approx=True)
        outs = []
        for g in range(gpb):
            acc = None
            for head in range(heads):
                lo = head * hd
                r0 = (g * heads + head) * n
                ho = jax.lax.dot_general(
                    p[r0:r0 + n, :], glT[lo:lo + hd, g * n:(g + 1) * n],
                    (((1,), (1,)), ((), ())),
                    preferred_element_type=jnp.float32)
                acc = ho if acc is None else acc + ho
            outs.append(acc)
        return jnp.concatenate(outs, axis=0) * (1.0 / heads) + cb_ref[...]

    xstk = x_ref[...].reshape(gpb * n, -1)
    h0 = jnp.maximum(
        jnp.dot(xstk, win_ref[...], preferred_element_type=jnp.float32)
        + bin_ref[...], 0.0)
    h1 = ln_relu(h0 + gat(h0, wlr1_ref, blr1t_ref, asc1_ref, lo1_ref,
                          hi1_ref, kap1_ref, cb1_ref),
                 g1_ref[...], be1_ref[...])
    h2 = ln_relu(h1 + gat(h1, wlr2_ref, blr2t_ref, asc2_ref, lo2_ref,
                          hi2_ref, kap2_ref, cb2_ref),
                 g2_ref[...], be2_ref[...])

    out2 = jax.lax.dot_general(wac_ref[...], h2, (((1,), (1,)), ((), ())),
                               preferred_element_type=jnp.float32)
    for g in range(gpb):
        j0 = g * n
        logits = out2[0:1, j0:j0 + n] + ba_ref[...]
        value = jnp.sum(out2[1:2, j0:j0 + n], axis=-1, keepdims=True) \
            * (1.0 / n) + bc_ref[...]
        out_ref[g] = jnp.concatenate(
            [logits, jnp.broadcast_to(value, (1, 128)),
             jnp.zeros((6, 128), jnp.float32)], axis=0)


def kernel(x, adj, w_in, b_in, wl1, bl1, wr1, br1, att1, cb1,
           wl2, bl2, wr2, br2, att2, cb2, g1, be1, g2, be2,
           wa, ba, wc, bc):
    b, n, f = x.shape
    heads, hd = att1.shape
    hidden = w_in.shape[1]
    hc = heads * hd

    wlr1 = jnp.concatenate([wl1, wr1], axis=1)
    wlr2 = jnp.concatenate([wl2, wr2], axis=1)
    blr1t = jnp.concatenate([bl1, br1], axis=1).T
    blr2t = jnp.concatenate([bl2, br2], axis=1).T

    def att_pack(att):
        a = att.reshape(hc, 1)
        asc = jnp.abs(0.8 * a)
        big = jnp.float32(1e30)
        lo = jnp.where(a < 0, -big, 0.0).astype(jnp.float32)
        hi = jnp.where(a < 0, 0.0, big).astype(jnp.float32)
        kap = 0.2 * a + 0.8 * jnp.where(a < 0, a, 0.0)
        return asc, lo, hi, kap

    asc1, lo1, hi1, kap1 = att_pack(att1)
    asc2, lo2, hi2, kap2 = att_pack(att2)
    wac = jnp.concatenate(
        [wa.T, wc.T, jnp.zeros((6, hidden), jnp.float32)], axis=0)

    gpb = 16 if b % 16 == 0 else (2 if b % 2 == 0 else 1)
    body = functools.partial(_acgnn_body, heads=heads, hd=hd, n=n, gpb=gpb)

    def fixed(a):
        return pl.BlockSpec(a.shape, lambda i: (0,) * a.ndim)

    smalls = (w_in, b_in,
              wlr1, blr1t, asc1, lo1, hi1, kap1, cb1, g1, be1,
              wlr2, blr2t, asc2, lo2, hi2, kap2, cb2, g2, be2,
              wac, ba, bc)

    out = pl.pallas_call(
        body,
        grid=(b // gpb,),
        out_shape=jax.ShapeDtypeStruct((b, 8, 128), jnp.float32),
        in_specs=[
            pl.BlockSpec((gpb, n, f), lambda i: (i, 0, 0)),
            pl.BlockSpec((gpb, n, n), lambda i: (i, 0, 0)),
        ] + [fixed(a) for a in smalls],
        out_specs=pl.BlockSpec((gpb, 8, 128), lambda i: (i, 0, 0)),
        compiler_params=pltpu.CompilerParams(
            dimension_semantics=("parallel",)),
    )(x, adj, *smalls)

    return out[:, 0, :n], out[:, 1, 0]


# speedup vs baseline: 1.1963x; 1.1963x over previous
import functools

import jax
import jax.numpy as jnp
from jax.experimental import pallas as pl
from jax.experimental.pallas import tpu as pltpu

_NEG = -1e30


def _acgnn_body(x_ref, adj_ref, win_ref, bin_ref,
                wlr1_ref, blr1t_ref, asc1_ref, lo1_ref, hi1_ref, kap1_ref,
                cb1_ref, g1_ref, be1_ref,
                wlr2_ref, blr2t_ref, asc2_ref, lo2_ref, hi2_ref, kap2_ref,
                cb2_ref, g2_ref, be2_ref,
                wac_ref, ba_ref, bc_ref, out_ref, *, heads, hd, n, gpb):
    hc = heads * hd

    adjbs = [jnp.where(adj_ref[g] > 0.0, 0.0, _NEG) for g in range(gpb)]

    def ln_relu(v, gamma, beta, eps=1e-5):
        mu = jnp.mean(v, axis=-1, keepdims=True)
        d = v - mu
        var = jnp.mean(d * d, axis=-1, keepdims=True)
        return jnp.maximum(d * jax.lax.rsqrt(var + eps) * gamma + beta, 0.0)

    def gat(hstk, wlr_ref, blrt_ref, asc_ref, lo_ref, hi_ref, kap_ref,
            cb_ref):
        gT = jax.lax.dot_general(wlr_ref[...], hstk, (((0,), (1,)), ((), ())),
                                 preferred_element_type=jnp.float32)
        gT = gT + blrt_ref[...]
        glT = gT[0:hc, :]
        grT = gT[hc:2 * hc, :]
        asc = asc_ref[...]
        glTs = (glT * asc).astype(jnp.bfloat16)
        grTs = (grT * asc).astype(jnp.bfloat16)
        clo = lo_ref[...].astype(jnp.bfloat16)
        chi = hi_ref[...].astype(jnp.bfloat16)
        gr3 = grTs[:, :, None]

        es = [None] * (gpb * heads)
        hgrp = 4
        for hp in range(heads // hgrp):
            accs = [None] * (hgrp * gpb)
            for c in range(hd):
                for hh in range(hgrp):
                    head = hgrp * hp + hh
                    sl = head * hd + c
                    l3 = clo[sl:sl + 1, :][:, :, None]
                    h3 = chi[sl:sl + 1, :][:, :, None]
                    for g in range(gpb):
                        j0 = g * n
                        u3 = gr3[sl:sl + 1, j0:j0 + n] \
                            + glTs[sl:sl + 1, None, j0:j0 + n]
                        r3 = jnp.clip(u3, l3, h3)
                        k = hh * gpb + g
                        accs[k] = r3 if accs[k] is None else accs[k] + r3
            for hh in range(hgrp):
                head = hgrp * hp + hh
                lo = head * hd
                for g in range(gpb):
                    j0 = g * n
                    lk = jax.lax.dot_general(
                        kap_ref[lo:lo + hd, :], glT[lo:lo + hd, j0:j0 + n],
                        (((0,), (0,)), ((), ())),
                        preferred_element_type=jnp.float32)
                    es[g * heads + head] = (
                        accs[hh * gpb + g][0].astype(jnp.float32)
                        + lk + adjbs[g])
        ee = jnp.concatenate(es, axis=0)
        ee = ee - jnp.max(ee, axis=-1, keepdims=True)
        p = jnp.exp(ee)
        p = p * pl.reciprocal(jnp.sum(p, axis=-1, keepdims=True),
                              approx=True)
        outs = []
        for g in range(gpb):
            acc = None
            for head in range(heads):
                lo = head * hd
                r0 = (g * heads + head) * n
                ho = jax.lax.dot_general(
                    p[r0:r0 + n, :], glT[lo:lo + hd, g * n:(g + 1) * n],
                    (((1,), (1,)), ((), ())),
                    preferred_element_type=jnp.float32)
                acc = ho if acc is None else acc + ho
            outs.append(acc)
        return jnp.concatenate(outs, axis=0) * (1.0 / heads) + cb_ref[...]

    xstk = x_ref[...].reshape(gpb * n, -1)
    h0 = jnp.maximum(
        jnp.dot(xstk, win_ref[...], preferred_element_type=jnp.float32)
        + bin_ref[...], 0.0)
    h1 = ln_relu(h0 + gat(h0, wlr1_ref, blr1t_ref, asc1_ref, lo1_ref,
                          hi1_ref, kap1_ref, cb1_ref),
                 g1_ref[...], be1_ref[...])
    h2 = ln_relu(h1 + gat(h1, wlr2_ref, blr2t_ref, asc2_ref, lo2_ref,
                          hi2_ref, kap2_ref, cb2_ref),
                 g2_ref[...], be2_ref[...])

    out2 = jax.lax.dot_general(wac_ref[...], h2, (((1,), (1,)), ((), ())),
                               preferred_element_type=jnp.float32)
    for g in range(gpb):
        j0 = g * n
        logits = out2[0:1, j0:j0 + n] + ba_ref[...]
        value = jnp.sum(out2[1:2, j0:j0 + n], axis=-1, keepdims=True) \
            * (1.0 / n) + bc_ref[...]
        out_ref[g] = jnp.concatenate(
            [logits, jnp.broadcast_to(value, (1, 128)),
             jnp.zeros((6, 128), jnp.float32)], axis=0)


def kernel(x, adj, w_in, b_in, wl1, bl1, wr1, br1, att1, cb1,
           wl2, bl2, wr2, br2, att2, cb2, g1, be1, g2, be2,
           wa, ba, wc, bc):
    b, n, f = x.shape
    heads, hd = att1.shape
    hidden = w_in.shape[1]
    hc = heads * hd

    wlr1 = jnp.concatenate([wl1, wr1], axis=1)
    wlr2 = jnp.concatenate([wl2, wr2], axis=1)
    blr1t = jnp.concatenate([bl1, br1], axis=1).T
    blr2t = jnp.concatenate([bl2, br2], axis=1).T

    def att_pack(att):
        a = att.reshape(hc, 1)
        asc = jnp.abs(0.8 * a)
        big = jnp.float32(1e30)
        lo = jnp.where(a < 0, -big, 0.0).astype(jnp.float32)
        hi = jnp.where(a < 0, 0.0, big).astype(jnp.float32)
        kap = 0.2 * a + 0.8 * jnp.where(a < 0, a, 0.0)
        return asc, lo, hi, kap

    asc1, lo1, hi1, kap1 = att_pack(att1)
    asc2, lo2, hi2, kap2 = att_pack(att2)
    wac = jnp.concatenate(
        [wa.T, wc.T, jnp.zeros((6, hidden), jnp.float32)], axis=0)

    gpb = 4 if b % 4 == 0 else (2 if b % 2 == 0 else 1)
    body = functools.partial(_acgnn_body, heads=heads, hd=hd, n=n, gpb=gpb)

    def fixed(a):
        return pl.BlockSpec(a.shape, lambda i: (0,) * a.ndim)

    smalls = (w_in, b_in,
              wlr1, blr1t, asc1, lo1, hi1, kap1, cb1, g1, be1,
              wlr2, blr2t, asc2, lo2, hi2, kap2, cb2, g2, be2,
              wac, ba, bc)

    out = pl.pallas_call(
        body,
        grid=(b // gpb,),
        out_shape=jax.ShapeDtypeStruct((b, 8, 128), jnp.float32),
        in_specs=[
            pl.BlockSpec((gpb, n, f), lambda i: (i, 0, 0)),
            pl.BlockSpec((gpb, n, n), lambda i: (i, 0, 0)),
        ] + [fixed(a) for a in smalls],
        out_specs=pl.BlockSpec((gpb, 8, 128), lambda i: (i, 0, 0)),
        compiler_params=pltpu.CompilerParams(
            dimension_semantics=("parallel",)),
    )(x, adj, *smalls)

    return out[:, 0, :n], out[:, 1, 0]


# speedup vs baseline: 1.2434x; 1.0394x over previous
import functools

import jax
import jax.numpy as jnp
from jax.experimental import pallas as pl
from jax.experimental.pallas import tpu as pltpu

_NEG = -1e30


def _acgnn_body(x_ref, adj_ref, win_ref, bin_ref,
                wlr1_ref, blr1t_ref, asc1_ref, lo1_ref, hi1_ref, kap1_ref,
                cb1_ref, g1_ref, be1_ref,
                wlr2_ref, blr2t_ref, asc2_ref, lo2_ref, hi2_ref, kap2_ref,
                cb2_ref, g2_ref, be2_ref,
                wac_ref, ba_ref, bc_ref, out_ref, *, heads, hd, n, gpb):
    hc = heads * hd

    adjbs = [jnp.where(adj_ref[g] > 0.0, 0.0, _NEG) for g in range(gpb)]

    def ln_relu(v, gamma, beta, eps=1e-5):
        mu = jnp.mean(v, axis=-1, keepdims=True)
        d = v - mu
        var = jnp.mean(d * d, axis=-1, keepdims=True)
        return jnp.maximum(d * jax.lax.rsqrt(var + eps) * gamma + beta, 0.0)

    def gat(hstk, wlr_ref, blrt_ref, asc_ref, lo_ref, hi_ref, kap_ref,
            cb_ref):
        gT = jax.lax.dot_general(wlr_ref[...], hstk, (((0,), (1,)), ((), ())),
                                 preferred_element_type=jnp.float32)
        gT = gT + blrt_ref[...]
        glT = gT[0:hc, :]
        grT = gT[hc:2 * hc, :]
        asc = asc_ref[...]
        glTs = (glT * asc).astype(jnp.bfloat16)
        grTs = (grT * asc).astype(jnp.bfloat16)
        clo = lo_ref[...].astype(jnp.bfloat16)
        chi = hi_ref[...].astype(jnp.bfloat16)
        gr3 = grTs[:, :, None]

        es = [None] * (gpb * heads)
        hgrp = 4
        for hp in range(heads // hgrp):
            accs = [None] * (hgrp * gpb)
            for c in range(hd):
                for hh in range(hgrp):
                    head = hgrp * hp + hh
                    sl = head * hd + c
                    l3 = clo[sl:sl + 1, :][:, :, None]
                    h3 = chi[sl:sl + 1, :][:, :, None]
                    for g in range(gpb):
                        j0 = g * n
                        u3 = gr3[sl:sl + 1, j0:j0 + n] \
                            + glTs[sl:sl + 1, None, j0:j0 + n]
                        r3 = jnp.clip(u3, l3, h3)
                        k = hh * gpb + g
                        accs[k] = r3 if accs[k] is None else accs[k] + r3
            for hh in range(hgrp):
                head = hgrp * hp + hh
                lo = head * hd
                for g in range(gpb):
                    j0 = g * n
                    lk = jax.lax.dot_general(
                        kap_ref[lo:lo + hd, :], glT[lo:lo + hd, j0:j0 + n],
                        (((0,), (0,)), ((), ())),
                        preferred_element_type=jnp.float32)
                    es[g * heads + head] = (
                        accs[hh * gpb + g][0].astype(jnp.float32)
                        + lk + adjbs[g])
        ee = jnp.concatenate(es, axis=0)
        ee = ee - jnp.max(ee, axis=-1, keepdims=True)
        p = jnp.exp(ee)
        p = p * pl.reciprocal(jnp.sum(p, axis=-1, keepdims=True),
                              approx=True)
        outs = []
        for g in range(gpb):
            acc = None
            for head in range(heads):
                lo = head * hd
                r0 = (g * heads + head) * n
                ho = jax.lax.dot_general(
                    p[r0:r0 + n, :], glT[lo:lo + hd, g * n:(g + 1) * n],
                    (((1,), (1,)), ((), ())),
                    preferred_element_type=jnp.float32)
                acc = ho if acc is None else acc + ho
            outs.append(acc)
        return jnp.concatenate(outs, axis=0) * (1.0 / heads) + cb_ref[...]

    xstk = x_ref[...].reshape(gpb * n, -1)
    h0 = jnp.maximum(
        jnp.dot(xstk, win_ref[...], preferred_element_type=jnp.float32)
        + bin_ref[...], 0.0)
    h1 = ln_relu(h0 + gat(h0, wlr1_ref, blr1t_ref, asc1_ref, lo1_ref,
                          hi1_ref, kap1_ref, cb1_ref),
                 g1_ref[...], be1_ref[...])
    h2 = ln_relu(h1 + gat(h1, wlr2_ref, blr2t_ref, asc2_ref, lo2_ref,
                          hi2_ref, kap2_ref, cb2_ref),
                 g2_ref[...], be2_ref[...])

    out2 = jax.lax.dot_general(wac_ref[...], h2, (((1,), (1,)), ((), ())),
                               preferred_element_type=jnp.float32)
    for g in range(gpb):
        j0 = g * n
        logits = out2[0:1, j0:j0 + n] + ba_ref[...]
        value = jnp.sum(out2[1:2, j0:j0 + n], axis=-1, keepdims=True) \
            * (1.0 / n) + bc_ref[...]
        out_ref[g] = jnp.concatenate(
            [logits, jnp.broadcast_to(value, (1, 128)),
             jnp.zeros((6, 128), jnp.float32)], axis=0)


def kernel(x, adj, w_in, b_in, wl1, bl1, wr1, br1, att1, cb1,
           wl2, bl2, wr2, br2, att2, cb2, g1, be1, g2, be2,
           wa, ba, wc, bc):
    b, n, f = x.shape
    heads, hd = att1.shape
    hidden = w_in.shape[1]
    hc = heads * hd

    wlr1 = jnp.concatenate([wl1, wr1], axis=1)
    wlr2 = jnp.concatenate([wl2, wr2], axis=1)
    blr1t = jnp.concatenate([bl1, br1], axis=1).T
    blr2t = jnp.concatenate([bl2, br2], axis=1).T

    def att_pack(att):
        a = att.reshape(hc, 1)
        asc = jnp.abs(0.8 * a)
        big = jnp.float32(1e30)
        lo = jnp.where(a < 0, -big, 0.0).astype(jnp.float32)
        hi = jnp.where(a < 0, 0.0, big).astype(jnp.float32)
        kap = 0.2 * a + 0.8 * jnp.where(a < 0, a, 0.0)
        return asc, lo, hi, kap

    asc1, lo1, hi1, kap1 = att_pack(att1)
    asc2, lo2, hi2, kap2 = att_pack(att2)
    wac = jnp.concatenate(
        [wa.T, wc.T, jnp.zeros((6, hidden), jnp.float32)], axis=0)

    gpb = 8 if b % 8 == 0 else (2 if b % 2 == 0 else 1)
    body = functools.partial(_acgnn_body, heads=heads, hd=hd, n=n, gpb=gpb)

    def fixed(a):
        return pl.BlockSpec(a.shape, lambda i: (0,) * a.ndim)

    smalls = (w_in, b_in,
              wlr1, blr1t, asc1, lo1, hi1, kap1, cb1, g1, be1,
              wlr2, blr2t, asc2, lo2, hi2, kap2, cb2, g2, be2,
              wac, ba, bc)

    out = pl.pallas_call(
        body,
        grid=(b // gpb,),
        out_shape=jax.ShapeDtypeStruct((b, 8, 128), jnp.float32),
        in_specs=[
            pl.BlockSpec((gpb, n, f), lambda i: (i, 0, 0)),
            pl.BlockSpec((gpb, n, n), lambda i: (i, 0, 0)),
        ] + [fixed(a) for a in smalls],
        out_specs=pl.BlockSpec((gpb, 8, 128), lambda i: (i, 0, 0)),
        compiler_params=pltpu.CompilerParams(
            dimension_semantics=("parallel",)),
    )(x, adj, *smalls)

    return out[:, 0, :n], out[:, 1, 0]
